# baseline (speedup 1.0000x reference)
import functools

import jax
import jax.numpy as jnp
from jax.experimental import pallas as pl
from jax.experimental.pallas import tpu as pltpu

K_NN = 3
Q_BLK = 2048


def _fused_kernel(p1_ref, p2_ref, w0_ref, b0_ref, w1_ref, b1_ref,
                  w2_ref, b2_ref, out_ref):
    p1 = p1_ref[0]
    p2t = p2_ref[0]
    xyz1 = p1[:, :3]
    feat1 = p1[:, 3:]
    xyz2t = p2t[:3, :]
    feat2t = p2t[3:, :]

    q = xyz1.shape[0]
    n2 = xyz2t.shape[1]

    a2 = jnp.sum(xyz1 * xyz1, axis=1, keepdims=True)
    b2row = jnp.sum(xyz2t * xyz2t, axis=0, keepdims=True)
    cross = jax.lax.dot_general(
        xyz1, xyz2t, (((1,), (0,)), ((), ())),
        preferred_element_type=jnp.float32)
    d2 = a2 + b2row - 2.0 * cross

    inf = jnp.float32(jnp.inf)
    nch = 8
    w = n2 // nch
    ch = [d2[:, j * w:(j + 1) * w] for j in range(nch)]
    l01 = jnp.minimum(ch[0], ch[1])
    h01 = jnp.maximum(ch[0], ch[1])
    r1 = jnp.minimum(l01, ch[2])
    hm = jnp.maximum(l01, ch[2])
    r2 = jnp.minimum(hm, h01)
    r3 = jnp.maximum(hm, h01)
    for j in range(3, nch):
        c = ch[j]
        u = jnp.maximum(r1, c)
        r1 = jnp.minimum(r1, c)
        v = jnp.maximum(r2, u)
        r2 = jnp.minimum(r2, u)
        r3 = jnp.minimum(r3, v)
    m1 = jnp.min(r1, axis=1, keepdims=True)
    e2 = jnp.minimum(jnp.where(r1 <= m1, inf, r1), r2)
    m2 = jnp.min(e2, axis=1, keepdims=True)
    e3 = jnp.minimum(jnp.where(r1 <= m2, inf, r1),
                     jnp.minimum(jnp.where(r2 <= m2, inf, r2), r3))
    m3 = jnp.min(e3, axis=1, keepdims=True)

    def _w(v):
        return jax.lax.rsqrt(jnp.maximum(v, 1e-12))
    inv_wsum = 1.0 / (_w(m1) + _w(m2) + _w(m3))
    s = jnp.where(d2 <= m3, _w(d2), 0.0)

    interp = jax.lax.dot_general(
        s.astype(jnp.bfloat16), feat2t.astype(jnp.bfloat16),
        (((1,), (1,)), ((), ())),
        preferred_element_type=jnp.float32) * inv_wsum

    h = jnp.maximum(
        jax.lax.dot_general(feat1, w0_ref[:13, :], (((1,), (0,)), ((), ())),
                            preferred_element_type=jnp.float32)
        + jax.lax.dot_general(interp, w0_ref[13:, :], (((1,), (0,)), ((), ())),
                              preferred_element_type=jnp.float32)
        + b0_ref[:], 0.0)
    h = jnp.maximum(jnp.dot(h, w1_ref[:], preferred_element_type=jnp.float32)
                    + b1_ref[:], 0.0)
    h = jnp.maximum(jnp.dot(h, w2_ref[:], preferred_element_type=jnp.float32)
                    + b2_ref[:], 0.0)
    out_ref[0] = jnp.concatenate([xyz1, h], axis=1)


@jax.jit
def kernel(points1, points2, W0, b0, W1, b1, W2, b2):
    B, N1, C1 = points1.shape
    _, N2, C2 = points2.shape
    grid = (B, N1 // Q_BLK)

    out = pl.pallas_call(
        _fused_kernel,
        grid=grid,
        in_specs=[
            pl.BlockSpec((1, Q_BLK, C1), lambda b, i: (b, i, 0)),
            pl.BlockSpec((1, C2, N2), lambda b, i: (b, 0, 0)),
            pl.BlockSpec(W0.shape, lambda b, i: (0, 0)),
            pl.BlockSpec((1, b0.shape[0]), lambda b, i: (0, 0)),
            pl.BlockSpec(W1.shape, lambda b, i: (0, 0)),
            pl.BlockSpec((1, b1.shape[0]), lambda b, i: (0, 0)),
            pl.BlockSpec(W2.shape, lambda b, i: (0, 0)),
            pl.BlockSpec((1, b2.shape[0]), lambda b, i: (0, 0)),
        ],
        out_specs=pl.BlockSpec((1, Q_BLK, 3 + W2.shape[1]),
                               lambda b, i: (b, i, 0)),
        out_shape=jax.ShapeDtypeStruct((B, N1, 3 + W2.shape[1]),
                                       jnp.float32),
        compiler_params=pltpu.CompilerParams(
            dimension_semantics=("parallel", "parallel")),
    )(points1, jnp.swapaxes(points2, 1, 2), W0, b0.reshape(1, -1),
      W1, b1.reshape(1, -1), W2, b2.reshape(1, -1))
    return out


# speedup vs baseline: 1.0323x; 1.0323x over previous
import functools

import jax
import jax.numpy as jnp
from jax.experimental import pallas as pl
from jax.experimental.pallas import tpu as pltpu

K_NN = 3
Q_BLK = 4096


def _fused_kernel(p1_ref, p2_ref, w0_ref, b0_ref, w1_ref, b1_ref,
                  w2_ref, b2_ref, out_ref):
    p1 = p1_ref[0]
    p2t = p2_ref[0]
    xyz1 = p1[:, :3]
    feat1 = p1[:, 3:]
    xyz2t = p2t[:3, :]
    feat2t = p2t[3:, :]

    q = xyz1.shape[0]
    n2 = xyz2t.shape[1]

    a2 = jnp.sum(xyz1 * xyz1, axis=1, keepdims=True)
    b2row = jnp.sum(xyz2t * xyz2t, axis=0, keepdims=True)
    cross = jax.lax.dot_general(
        xyz1, xyz2t, (((1,), (0,)), ((), ())),
        preferred_element_type=jnp.float32)
    d2 = a2 + b2row - 2.0 * cross

    inf = jnp.float32(jnp.inf)
    nch = 8
    w = n2 // nch
    ch = [d2[:, j * w:(j + 1) * w] for j in range(nch)]
    l01 = jnp.minimum(ch[0], ch[1])
    h01 = jnp.maximum(ch[0], ch[1])
    r1 = jnp.minimum(l01, ch[2])
    hm = jnp.maximum(l01, ch[2])
    r2 = jnp.minimum(hm, h01)
    r3 = jnp.maximum(hm, h01)
    for j in range(3, nch):
        c = ch[j]
        u = jnp.maximum(r1, c)
        r1 = jnp.minimum(r1, c)
        v = jnp.maximum(r2, u)
        r2 = jnp.minimum(r2, u)
        r3 = jnp.minimum(r3, v)
    m1 = jnp.min(r1, axis=1, keepdims=True)
    e2 = jnp.minimum(jnp.where(r1 <= m1, inf, r1), r2)
    m2 = jnp.min(e2, axis=1, keepdims=True)
    e3 = jnp.minimum(jnp.where(r1 <= m2, inf, r1),
                     jnp.minimum(jnp.where(r2 <= m2, inf, r2), r3))
    m3 = jnp.min(e3, axis=1, keepdims=True)

    def _w(v):
        return jax.lax.rsqrt(jnp.maximum(v, 1e-12))
    inv_wsum = 1.0 / (_w(m1) + _w(m2) + _w(m3))
    s = jnp.where(d2 <= m3, _w(d2), 0.0)

    interp = jax.lax.dot_general(
        s.astype(jnp.bfloat16), feat2t.astype(jnp.bfloat16),
        (((1,), (1,)), ((), ())),
        preferred_element_type=jnp.float32) * inv_wsum

    h = jnp.maximum(
        jax.lax.dot_general(feat1, w0_ref[:13, :], (((1,), (0,)), ((), ())),
                            preferred_element_type=jnp.float32)
        + jax.lax.dot_general(interp, w0_ref[13:, :], (((1,), (0,)), ((), ())),
                              preferred_element_type=jnp.float32)
        + b0_ref[:], 0.0)
    h = jnp.maximum(jnp.dot(h, w1_ref[:], preferred_element_type=jnp.float32)
                    + b1_ref[:], 0.0)
    h = jnp.maximum(jnp.dot(h, w2_ref[:], preferred_element_type=jnp.float32)
                    + b2_ref[:], 0.0)
    out_ref[0] = jnp.concatenate([xyz1, h], axis=1)


@jax.jit
def kernel(points1, points2, W0, b0, W1, b1, W2, b2):
    B, N1, C1 = points1.shape
    _, N2, C2 = points2.shape
    grid = (B, N1 // Q_BLK)

    out = pl.pallas_call(
        _fused_kernel,
        grid=grid,
        in_specs=[
            pl.BlockSpec((1, Q_BLK, C1), lambda b, i: (b, i, 0)),
            pl.BlockSpec((1, C2, N2), lambda b, i: (b, 0, 0)),
            pl.BlockSpec(W0.shape, lambda b, i: (0, 0)),
            pl.BlockSpec((1, b0.shape[0]), lambda b, i: (0, 0)),
            pl.BlockSpec(W1.shape, lambda b, i: (0, 0)),
            pl.BlockSpec((1, b1.shape[0]), lambda b, i: (0, 0)),
            pl.BlockSpec(W2.shape, lambda b, i: (0, 0)),
            pl.BlockSpec((1, b2.shape[0]), lambda b, i: (0, 0)),
        ],
        out_specs=pl.BlockSpec((1, Q_BLK, 3 + W2.shape[1]),
                               lambda b, i: (b, i, 0)),
        out_shape=jax.ShapeDtypeStruct((B, N1, 3 + W2.shape[1]),
                                       jnp.float32),
        compiler_params=pltpu.CompilerParams(
            dimension_semantics=("parallel", "parallel")),
    )(points1, jnp.swapaxes(points2, 1, 2), W0, b0.reshape(1, -1),
      W1, b1.reshape(1, -1), W2, b2.reshape(1, -1))
    return out


# speedup vs baseline: 1.0535x; 1.0206x over previous
import functools

import jax
import jax.numpy as jnp
from jax.experimental import pallas as pl
from jax.experimental.pallas import tpu as pltpu

K_NN = 3
Q_BLK = 8192


def _fused_kernel(p1_ref, p2_ref, w0_ref, b0_ref, w1_ref, b1_ref,
                  w2_ref, b2_ref, out_ref):
    p1 = p1_ref[0]
    p2t = p2_ref[0]
    xyz1 = p1[:, :3]
    feat1 = p1[:, 3:]
    xyz2t = p2t[:3, :]
    feat2t = p2t[3:, :]

    q = xyz1.shape[0]
    n2 = xyz2t.shape[1]

    a2 = jnp.sum(xyz1 * xyz1, axis=1, keepdims=True)
    b2row = jnp.sum(xyz2t * xyz2t, axis=0, keepdims=True)
    cross = jax.lax.dot_general(
        xyz1, xyz2t, (((1,), (0,)), ((), ())),
        preferred_element_type=jnp.float32)
    d2 = a2 + b2row - 2.0 * cross

    inf = jnp.float32(jnp.inf)
    m1 = jnp.min(d2, axis=1, keepdims=True)
    m2 = jnp.min(jnp.where(d2 <= m1, inf, d2), axis=1, keepdims=True)
    m3 = jnp.min(jnp.where(d2 <= m2, inf, d2), axis=1, keepdims=True)

    def _w(v):
        return jax.lax.rsqrt(jnp.maximum(v, 1e-12))
    inv_wsum = 1.0 / (_w(m1) + _w(m2) + _w(m3))
    s = jnp.where(d2 <= m3, _w(d2), 0.0)

    interp = jax.lax.dot_general(
        s.astype(jnp.bfloat16), feat2t.astype(jnp.bfloat16),
        (((1,), (1,)), ((), ())),
        preferred_element_type=jnp.float32) * inv_wsum

    h = jnp.maximum(
        jax.lax.dot_general(feat1, w0_ref[:13, :], (((1,), (0,)), ((), ())),
                            preferred_element_type=jnp.float32)
        + jax.lax.dot_general(interp, w0_ref[13:, :], (((1,), (0,)), ((), ())),
                              preferred_element_type=jnp.float32)
        + b0_ref[:], 0.0)
    h = jnp.maximum(jnp.dot(h, w1_ref[:], preferred_element_type=jnp.float32)
                    + b1_ref[:], 0.0)
    h = jnp.maximum(jnp.dot(h, w2_ref[:], preferred_element_type=jnp.float32)
                    + b2_ref[:], 0.0)
    out_ref[0] = jnp.concatenate([xyz1, h], axis=1)


@jax.jit
def kernel(points1, points2, W0, b0, W1, b1, W2, b2):
    B, N1, C1 = points1.shape
    _, N2, C2 = points2.shape
    grid = (B, N1 // Q_BLK)

    out = pl.pallas_call(
        _fused_kernel,
        grid=grid,
        in_specs=[
            pl.BlockSpec((1, Q_BLK, C1), lambda b, i: (b, i, 0)),
            pl.BlockSpec((1, C2, N2), lambda b, i: (b, 0, 0)),
            pl.BlockSpec(W0.shape, lambda b, i: (0, 0)),
            pl.BlockSpec((1, b0.shape[0]), lambda b, i: (0, 0)),
            pl.BlockSpec(W1.shape, lambda b, i: (0, 0)),
            pl.BlockSpec((1, b1.shape[0]), lambda b, i: (0, 0)),
            pl.BlockSpec(W2.shape, lambda b, i: (0, 0)),
            pl.BlockSpec((1, b2.shape[0]), lambda b, i: (0, 0)),
        ],
        out_specs=pl.BlockSpec((1, Q_BLK, 3 + W2.shape[1]),
                               lambda b, i: (b, i, 0)),
        out_shape=jax.ShapeDtypeStruct((B, N1, 3 + W2.shape[1]),
                                       jnp.float32),
        compiler_params=pltpu.CompilerParams(
            dimension_semantics=("parallel", "parallel")),
    )(points1, jnp.swapaxes(points2, 1, 2), W0, b0.reshape(1, -1),
      W1, b1.reshape(1, -1), W2, b2.reshape(1, -1))
    return out


# speedup vs baseline: 1.0544x; 1.0008x over previous
import jax
import jax.numpy as jnp
from jax.experimental import pallas as pl
from jax.experimental.pallas import tpu as pltpu

K_NN = 3
Q_BLK = 8192


def _fused_kernel(p1_ref, p2_ref, w0_ref, b0_ref, w1_ref, b1_ref,
                  w2_ref, b2_ref, out_ref):
    p1 = p1_ref[0]
    p2t = p2_ref[0]
    xyz1 = p1[:, :3]
    feat1 = p1[:, 3:]
    xyz2t = p2t[:3, :]
    feat2t = p2t[3:, :]

    a2 = jnp.sum(xyz1 * xyz1, axis=1, keepdims=True)
    b2row = jnp.sum(xyz2t * xyz2t, axis=0, keepdims=True)
    cross = jax.lax.dot_general(
        xyz1, xyz2t, (((1,), (0,)), ((), ())),
        preferred_element_type=jnp.float32)
    d2 = a2 + b2row - 2.0 * cross

    inf = jnp.float32(jnp.inf)
    m1 = jnp.min(d2, axis=1, keepdims=True)
    m2 = jnp.min(jnp.where(d2 <= m1, inf, d2), axis=1, keepdims=True)
    m3 = jnp.min(jnp.where(d2 <= m2, inf, d2), axis=1, keepdims=True)

    def _w(v):
        return jax.lax.rsqrt(jnp.maximum(v, 1e-12))
    inv_wsum = 1.0 / (_w(m1) + _w(m2) + _w(m3))
    s = jnp.where(d2 <= m3, _w(d2), 0.0)

    interp = jax.lax.dot_general(
        s.astype(jnp.bfloat16), feat2t.astype(jnp.bfloat16),
        (((1,), (1,)), ((), ())),
        preferred_element_type=jnp.float32) * inv_wsum

    h = jnp.maximum(
        jax.lax.dot_general(feat1, w0_ref[:13, :], (((1,), (0,)), ((), ())),
                            preferred_element_type=jnp.float32)
        + jax.lax.dot_general(interp, w0_ref[13:, :], (((1,), (0,)), ((), ())),
                              preferred_element_type=jnp.float32)
        + b0_ref[:], 0.0)
    h = jnp.maximum(jnp.dot(h, w1_ref[:], preferred_element_type=jnp.float32)
                    + b1_ref[:], 0.0)
    h = jnp.maximum(jnp.dot(h, w2_ref[:], preferred_element_type=jnp.float32)
                    + b2_ref[:], 0.0)
    out_ref[0] = jnp.concatenate([xyz1, h], axis=1)


@jax.jit
def kernel(points1, points2, W0, b0, W1, b1, W2, b2):
    B, N1, C1 = points1.shape
    _, N2, C2 = points2.shape
    grid = (B, N1 // Q_BLK)

    out = pl.pallas_call(
        _fused_kernel,
        grid=grid,
        in_specs=[
            pl.BlockSpec((1, Q_BLK, C1), lambda b, i: (b, i, 0)),
            pl.BlockSpec((1, C2, N2), lambda b, i: (b, 0, 0)),
            pl.BlockSpec(W0.shape, lambda b, i: (0, 0)),
            pl.BlockSpec((1, b0.shape[0]), lambda b, i: (0, 0)),
            pl.BlockSpec(W1.shape, lambda b, i: (0, 0)),
            pl.BlockSpec((1, b1.shape[0]), lambda b, i: (0, 0)),
            pl.BlockSpec(W2.shape, lambda b, i: (0, 0)),
            pl.BlockSpec((1, b2.shape[0]), lambda b, i: (0, 0)),
        ],
        out_specs=pl.BlockSpec((1, Q_BLK, 3 + W2.shape[1]),
                               lambda b, i: (b, i, 0)),
        out_shape=jax.ShapeDtypeStruct((B, N1, 3 + W2.shape[1]),
                                       jnp.float32),
        compiler_params=pltpu.CompilerParams(
            dimension_semantics=("parallel", "parallel")),
    )(points1, jnp.swapaxes(points2, 1, 2), W0, b0.reshape(1, -1),
      W1, b1.reshape(1, -1), W2, b2.reshape(1, -1))
    return out
